# Initial kernel scaffold; baseline (speedup 1.0000x reference)
import jax
import jax.numpy as jnp
from jax.experimental import pallas as pl


def kernel(x):
    raise NotImplementedError("write your pallas kernel here")


# speedup vs baseline: 3.1350x; 3.1350x over previous
import functools

import jax
import jax.numpy as jnp
from jax.experimental import pallas as pl
from jax.experimental.pallas import tpu as pltpu

NUM_CLASSES_K = 21
NCLS = 20
NPRIORS = 5000
OUT_SZ = 19
KEEP = 200
IOU_T = 0.45
NEG_INF = float("-inf")


def _decode(xt):
    loc0 = xt[0:1]
    loc1 = xt[1:2]
    loc2 = xt[2:3]
    loc3 = xt[3:4]
    p0 = xt[25:26]
    p1 = xt[26:27]
    p2 = xt[27:28]
    p3 = xt[28:29]
    v0 = xt[29:30]
    v1 = xt[30:31]
    v2 = xt[31:32]
    v3 = xt[32:33]
    pw = p2 - p0
    ph = p3 - p1
    pcx = 0.5 * (p2 + p0)
    pcy = 0.5 * (p3 + p1)
    cx = loc0 * pw * v0 + pcx
    cy = loc1 * pw * v1 + pcy
    w = jnp.exp(loc2 * v2) * pw
    h = jnp.exp(loc3 * v3) * ph
    x1 = jnp.clip(cx - 0.5 * w, 0.0, 1.0)
    y1 = jnp.clip(cy - 0.5 * h, 0.0, 1.0)
    x2 = jnp.clip(cx + 0.5 * w, 0.0, 1.0)
    y2 = jnp.clip(cy + 0.5 * h, 0.0, 1.0)
    area = (x2 - x1) * (y2 - y1)
    return x1, y1, x2, y2, area


def _paint_step(score, row, xmin, ymin, xmax, ymax, ok, acc):
    yy = jax.lax.broadcasted_iota(
        jnp.int32, (OUT_SZ, OUT_SZ, NUM_CLASSES_K), 0).astype(jnp.float32)
    xx = jax.lax.broadcasted_iota(
        jnp.int32, (OUT_SZ, OUT_SZ, NUM_CLASSES_K), 1).astype(jnp.float32)
    ch = jax.lax.broadcasted_iota(jnp.int32, (OUT_SZ, OUT_SZ, NUM_CLASSES_K), 2)
    cell = (yy >= ymin) & (yy < ymax) & (xx >= xmin) & (xx < xmax) & (ch == row)
    val = jnp.where(cell & ok, score, jnp.float32(jnp.inf))
    return jnp.minimum(acc, val)


def _nms_kernel(x_ref, out_ref):
    xt = x_ref[0]
    x1, y1, x2, y2, area = _decode(xt)
    confs = xt[5:25]
    sw0 = jnp.where(confs >= 0.6, confs, NEG_INF)
    iota_n = jax.lax.broadcasted_iota(jnp.int32, (1, NPRIORS), 1)

    def nms_body(k, carry):
        sw, ks, kx1, ky1, kx2, ky2 = carry
        m = jnp.max(sw, axis=1, keepdims=True)
        ok = m > NEG_INF
        idx = jnp.min(jnp.where(sw == m, iota_n, jnp.int32(2**30)),
                      axis=1, keepdims=True)
        pick = (iota_n == idx)
        px1 = jnp.sum(jnp.where(pick, x1, 0.0), axis=1, keepdims=True)
        py1 = jnp.sum(jnp.where(pick, y1, 0.0), axis=1, keepdims=True)
        px2 = jnp.sum(jnp.where(pick, x2, 0.0), axis=1, keepdims=True)
        py2 = jnp.sum(jnp.where(pick, y2, 0.0), axis=1, keepdims=True)
        parea = (px2 - px1) * (py2 - py1)
        xx1 = jnp.maximum(px1, x1)
        yy1 = jnp.maximum(py1, y1)
        xx2 = jnp.minimum(px2, x2)
        yy2 = jnp.minimum(py2, y2)
        inter = jnp.maximum(0.0, xx2 - xx1) * jnp.maximum(0.0, yy2 - yy1)
        union = jnp.maximum(parea + area - inter, 1e-12)
        iou = inter / union
        supp = (iou > IOU_T) | pick
        sw = jnp.where(ok & supp, NEG_INF, sw)
        mk = jnp.where(ok, m, NEG_INF)
        col = (jax.lax.broadcasted_iota(jnp.int32, (NCLS, KEEP), 1) == k)
        ks = jnp.where(col, mk, ks)
        kx1 = jnp.where(col, px1, kx1)
        ky1 = jnp.where(col, py1, ky1)
        kx2 = jnp.where(col, px2, kx2)
        ky2 = jnp.where(col, py2, ky2)
        return sw, ks, kx1, ky1, kx2, ky2

    zed = jnp.zeros((NCLS, KEEP), jnp.float32)
    ks0 = jnp.full((NCLS, KEEP), NEG_INF, jnp.float32)
    _, ks, kx1, ky1, kx2, ky2 = jax.lax.fori_loop(
        0, KEEP, nms_body, (sw0, ks0, zed, zed, zed, zed))

    scale = jnp.float32(OUT_SZ)
    rx1 = jnp.round(kx1 * scale)
    ry1 = jnp.round(ky1 * scale)
    rx2 = jnp.round(kx2 * scale)
    ry2 = jnp.round(ky2 * scale)

    flat = (jax.lax.broadcasted_iota(jnp.int32, (NCLS, KEEP), 0) * 256
            + jax.lax.broadcasted_iota(jnp.int32, (NCLS, KEEP), 1))

    def ext_body(k, carry):
        es, acc = carry
        m = jnp.max(es)
        ok = m > NEG_INF
        fk = jnp.min(jnp.where(es == m, flat, jnp.int32(2**30)))
        pick = (flat == fk)
        bx1 = jnp.sum(jnp.where(pick, rx1, 0.0))
        by1 = jnp.sum(jnp.where(pick, ry1, 0.0))
        bx2 = jnp.sum(jnp.where(pick, rx2, 0.0))
        by2 = jnp.sum(jnp.where(pick, ry2, 0.0))
        row = fk // 256
        acc = _paint_step(m, row, bx1, by1, bx2, by2, ok, acc)
        es = jnp.where(pick, NEG_INF, es)
        return es, acc

    acc0 = jnp.full((OUT_SZ, OUT_SZ, NUM_CLASSES_K), jnp.inf, jnp.float32)
    _, acc = jax.lax.fori_loop(0, KEEP, ext_body, (ks, acc0))
    out_ref[0] = jnp.where(jnp.isfinite(acc), acc, 0.0)


@functools.partial(jax.jit, static_argnames=("interpret",))
def _run(x, interpret=False):
    xt = jnp.transpose(x, (0, 2, 1))
    b = x.shape[0]
    return pl.pallas_call(
        _nms_kernel,
        grid=(b,),
        in_specs=[pl.BlockSpec((1, 33, NPRIORS), lambda i: (i, 0, 0))],
        out_specs=pl.BlockSpec((1, OUT_SZ, OUT_SZ, NUM_CLASSES_K),
                               lambda i: (i, 0, 0, 0)),
        out_shape=jax.ShapeDtypeStruct((b, OUT_SZ, OUT_SZ, NUM_CLASSES_K),
                                       jnp.float32),
        interpret=interpret,
    )(xt)


def kernel(x):
    return _run(x)


# speedup vs baseline: 6.1818x; 1.9718x over previous
import functools

import jax
import jax.numpy as jnp
from jax.experimental import pallas as pl
from jax.experimental.pallas import tpu as pltpu

NUM_CLASSES_K = 21
NCLS = 20
NPRIORS = 5000
OUT_SZ = 19
KEEP = 200
IOU_T = 0.45
NEG_INF = float("-inf")


def _decode(xt):
    loc0 = xt[0:1]
    loc1 = xt[1:2]
    loc2 = xt[2:3]
    loc3 = xt[3:4]
    p0 = xt[25:26]
    p1 = xt[26:27]
    p2 = xt[27:28]
    p3 = xt[28:29]
    v0 = xt[29:30]
    v1 = xt[30:31]
    v2 = xt[31:32]
    v3 = xt[32:33]
    pw = p2 - p0
    ph = p3 - p1
    pcx = 0.5 * (p2 + p0)
    pcy = 0.5 * (p3 + p1)
    cx = loc0 * pw * v0 + pcx
    cy = loc1 * pw * v1 + pcy
    w = jnp.exp(loc2 * v2) * pw
    h = jnp.exp(loc3 * v3) * ph
    x1 = jnp.clip(cx - 0.5 * w, 0.0, 1.0)
    y1 = jnp.clip(cy - 0.5 * h, 0.0, 1.0)
    x2 = jnp.clip(cx + 0.5 * w, 0.0, 1.0)
    y2 = jnp.clip(cy + 0.5 * h, 0.0, 1.0)
    area = (x2 - x1) * (y2 - y1)
    return x1, y1, x2, y2, area


def _paint_step(score, row, xmin, ymin, xmax, ymax, ok, acc):
    yy = jax.lax.broadcasted_iota(
        jnp.int32, (OUT_SZ, OUT_SZ, NUM_CLASSES_K), 0).astype(jnp.float32)
    xx = jax.lax.broadcasted_iota(
        jnp.int32, (OUT_SZ, OUT_SZ, NUM_CLASSES_K), 1).astype(jnp.float32)
    ch = jax.lax.broadcasted_iota(jnp.int32, (OUT_SZ, OUT_SZ, NUM_CLASSES_K), 2)
    cell = (yy >= ymin) & (yy < ymax) & (xx >= xmin) & (xx < xmax) & (ch == row)
    val = jnp.where(cell & ok, score, jnp.float32(jnp.inf))
    return jnp.minimum(acc, val)


def _nms_kernel(x_ref, out_ref):
    xt = x_ref[0]
    x1, y1, x2, y2, area = _decode(xt)
    confs = xt[5:25]
    sw0 = jnp.where(confs >= 0.6, confs, NEG_INF)
    iota_n = jax.lax.broadcasted_iota(jnp.int32, (1, NPRIORS), 1)

    def nms_body(k, carry):
        sw, ks, kx1, ky1, kx2, ky2 = carry
        m = jnp.max(sw, axis=1, keepdims=True)
        ok = m > NEG_INF
        idx = jnp.min(jnp.where(sw == m, iota_n, jnp.int32(2**30)),
                      axis=1, keepdims=True)
        pick = (iota_n == idx)
        px1 = jnp.sum(jnp.where(pick, x1, 0.0), axis=1, keepdims=True)
        py1 = jnp.sum(jnp.where(pick, y1, 0.0), axis=1, keepdims=True)
        px2 = jnp.sum(jnp.where(pick, x2, 0.0), axis=1, keepdims=True)
        py2 = jnp.sum(jnp.where(pick, y2, 0.0), axis=1, keepdims=True)
        parea = (px2 - px1) * (py2 - py1)
        xx1 = jnp.maximum(px1, x1)
        yy1 = jnp.maximum(py1, y1)
        xx2 = jnp.minimum(px2, x2)
        yy2 = jnp.minimum(py2, y2)
        inter = jnp.maximum(0.0, xx2 - xx1) * jnp.maximum(0.0, yy2 - yy1)
        union = jnp.maximum(parea + area - inter, 1e-12)
        iou = inter / union
        supp = (iou > IOU_T) | pick
        sw = jnp.where(ok & supp, NEG_INF, sw)
        mk = jnp.where(ok, m, NEG_INF)
        col = (jax.lax.broadcasted_iota(jnp.int32, (NCLS, KEEP), 1) == k)
        ks = jnp.where(col, mk, ks)
        kx1 = jnp.where(col, px1, kx1)
        ky1 = jnp.where(col, py1, ky1)
        kx2 = jnp.where(col, px2, kx2)
        ky2 = jnp.where(col, py2, ky2)
        return sw, ks, kx1, ky1, kx2, ky2

    zed = jnp.zeros((NCLS, KEEP), jnp.float32)
    ks0 = jnp.full((NCLS, KEEP), NEG_INF, jnp.float32)
    _, ks, kx1, ky1, kx2, ky2 = jax.lax.fori_loop(
        0, KEEP, nms_body, (sw0, ks0, zed, zed, zed, zed))

    scale = jnp.float32(OUT_SZ)
    rx1 = jnp.round(kx1 * scale)
    ry1 = jnp.round(ky1 * scale)
    rx2 = jnp.round(kx2 * scale)
    ry2 = jnp.round(ky2 * scale)

    flat = (jax.lax.broadcasted_iota(jnp.int32, (NCLS, KEEP), 0) * 256
            + jax.lax.broadcasted_iota(jnp.int32, (NCLS, KEEP), 1))

    def ext_body(k, carry):
        es, acc = carry
        m = jnp.max(es)
        ok = m > NEG_INF
        fk = jnp.min(jnp.where(es == m, flat, jnp.int32(2**30)))
        pick = (flat == fk)
        bx1 = jnp.sum(jnp.where(pick, rx1, 0.0))
        by1 = jnp.sum(jnp.where(pick, ry1, 0.0))
        bx2 = jnp.sum(jnp.where(pick, rx2, 0.0))
        by2 = jnp.sum(jnp.where(pick, ry2, 0.0))
        row = fk // 256
        acc = _paint_step(m, row, bx1, by1, bx2, by2, ok, acc)
        es = jnp.where(pick, NEG_INF, es)
        return es, acc

    acc0 = jnp.full((OUT_SZ, OUT_SZ, NUM_CLASSES_K), jnp.inf, jnp.float32)
    _, acc = jax.lax.fori_loop(0, KEEP, ext_body, (ks, acc0))
    out_ref[0] = jnp.where(jnp.isfinite(acc), acc, 0.0)


@functools.partial(jax.jit, static_argnames=("interpret",))
def _run(x, interpret=False):
    xt = jnp.transpose(x, (0, 2, 1))
    b = x.shape[0]
    return pl.pallas_call(
        _nms_kernel,
        grid=(b,),
        in_specs=[pl.BlockSpec((1, 33, NPRIORS), lambda i: (i, 0, 0))],
        out_specs=pl.BlockSpec((1, OUT_SZ, OUT_SZ, NUM_CLASSES_K),
                               lambda i: (i, 0, 0, 0)),
        out_shape=jax.ShapeDtypeStruct((b, OUT_SZ, OUT_SZ, NUM_CLASSES_K),
                                       jnp.float32),
        interpret=interpret,
    )(xt)


CAP = 64
TGT = 48.0
JAC_IT = 8
CHUNK = 1000


def _fast_kernel(x_ref, out_ref, flag_ref):
    xt = x_ref[0]
    x1, y1, x2, y2, _ = _decode(xt)
    confs = xt[5:25]
    one = jnp.float32(1.0)
    zero = jnp.float32(0.0)
    inf = jnp.float32(jnp.inf)

    tot_c = jnp.sum(jnp.where(confs >= 0.6, one, zero), axis=1, keepdims=True)

    def bis_body(_, lh):
        lo, hi = lh
        mid = 0.5 * (lo + hi)
        cnt = jnp.sum(jnp.where(confs >= mid, one, zero), axis=1, keepdims=True)
        ge = cnt >= TGT
        return jnp.where(ge, mid, lo), jnp.where(ge, hi, mid)

    lo0 = jnp.full((NCLS, 1), 0.6, jnp.float32)
    hi0 = jnp.full((NCLS, 1), 1.0, jnp.float32)
    tau, _ = jax.lax.fori_loop(0, 22, bis_body, (lo0, hi0))

    mask = confs >= tau
    maskf = jnp.where(mask, one, zero)
    cnt_c = jnp.sum(maskf, axis=1, keepdims=True)

    m3 = maskf.reshape(NCLS, 25, 200)
    st200 = jnp.where(
        jax.lax.broadcasted_iota(jnp.int32, (200, 200), 0)
        < jax.lax.broadcasted_iota(jnp.int32, (200, 200), 1), one, zero)
    pw = jax.lax.dot_general(m3, st200, (((2,), (0,)), ((), ())),
                             preferred_element_type=jnp.float32,
            precision=jax.lax.Precision.HIGHEST)
    tots = jnp.sum(m3, axis=2)
    st25 = jnp.where(
        jax.lax.broadcasted_iota(jnp.int32, (25, 25), 0)
        < jax.lax.broadcasted_iota(jnp.int32, (25, 25), 1), one, zero)
    offs = jax.lax.dot_general(tots, st25, (((1,), (0,)), ((), ())),
                               preferred_element_type=jnp.float32,
            precision=jax.lax.Precision.HIGHEST)
    pos = (pw + offs[:, :, None]).reshape(NCLS, NPRIORS)

    kio = jax.lax.broadcasted_iota(jnp.int32, (1, CAP, 1), 1).astype(jnp.float32)
    comp = jnp.zeros((NCLS, CAP, 5), jnp.float32)
    for t in range(NPRIORS // CHUNK):
        sl = slice(t * CHUNK, (t + 1) * CHUNK)
        posm = pos[:, sl][:, None, :]
        mk = maskf[:, sl][:, None, :]
        ohc = jnp.where((posm == kio) & (mk > 0.5), one, zero)
        vals = jnp.concatenate([
            confs[:, sl][:, :, None],
            jnp.broadcast_to(x1[:, sl], (NCLS, CHUNK))[:, :, None],
            jnp.broadcast_to(y1[:, sl], (NCLS, CHUNK))[:, :, None],
            jnp.broadcast_to(x2[:, sl], (NCLS, CHUNK))[:, :, None],
            jnp.broadcast_to(y2[:, sl], (NCLS, CHUNK))[:, :, None],
        ], axis=2)
        comp = comp + jax.lax.dot_general(
            ohc, vals, (((2,), (1,)), ((0,), (0,))),
            preferred_element_type=jnp.float32,
            precision=jax.lax.Precision.HIGHEST)

    cs = comp[:, :, 0]
    cx1 = comp[:, :, 1]
    cy1 = comp[:, :, 2]
    cx2 = comp[:, :, 3]
    cy2 = comp[:, :, 4]

    slot = jax.lax.broadcasted_iota(jnp.int32, (NCLS, CAP), 1).astype(jnp.float32)
    validf = jnp.where(slot < cnt_c, one, zero)
    valid = validf > 0.5
    vpair = (validf[:, :, None] > 0.5) & (validf[:, None, :] > 0.5)

    ai = (cx2 - cx1) * (cy2 - cy1)
    xx1 = jnp.maximum(cx1[:, :, None], cx1[:, None, :])
    yy1 = jnp.maximum(cy1[:, :, None], cy1[:, None, :])
    xx2 = jnp.minimum(cx2[:, :, None], cx2[:, None, :])
    yy2 = jnp.minimum(cy2[:, :, None], cy2[:, None, :])
    inter = jnp.maximum(zero, xx2 - xx1) * jnp.maximum(zero, yy2 - yy1)
    union = jnp.maximum(ai[:, :, None] + ai[:, None, :] - inter, 1e-12)
    iou = inter / union
    si = jax.lax.broadcasted_iota(jnp.int32, (1, CAP, CAP), 1)
    sj = jax.lax.broadcasted_iota(jnp.int32, (1, CAP, CAP), 2)
    higher = ((cs[:, :, None] > cs[:, None, :])
              | ((cs[:, :, None] == cs[:, None, :]) & (si < sj)))
    sm = jnp.where(higher & (iou > IOU_T) & vpair, one, zero)

    def jac(k):
        sup = jax.lax.dot_general(sm, k, (((1,), (1,)), ((0,), (0,))),
                                  preferred_element_type=jnp.float32,
            precision=jax.lax.Precision.HIGHEST)
        return jnp.where((sup < 0.5) & valid, one, zero)

    kcur = jax.lax.fori_loop(0, JAC_IT, lambda _, k: jac(k), validf)
    knext = jac(kcur)
    fp_ok = jnp.min(jnp.where(knext == kcur, one, zero)) > 0.5
    keep = knext > 0.5
    total_kept = jnp.sum(knext)

    def b2_body(_, lh):
        lo, hi = lh
        mid = 0.5 * (lo + hi)
        cnt = jnp.sum(jnp.where(keep & (cs >= mid), one, zero))
        ge = cnt >= jnp.float32(KEEP)
        return jnp.where(ge, mid, lo), jnp.where(ge, hi, mid)

    glo, _ = jax.lax.fori_loop(
        0, 26, b2_body, (jnp.float32(0.5), jnp.float32(1.0)))
    tsel = keep & (cs >= glo)
    b = jnp.min(jnp.where(tsel, cs, inf))
    above = tsel & (cs > b)
    na = jnp.sum(jnp.where(above, one, zero))
    boundary = tsel & (cs == b)
    q = jnp.float32(KEEP) - na
    flat = (jax.lax.broadcasted_iota(jnp.int32, (NCLS, CAP), 0) * CAP
            + jax.lax.broadcasted_iota(jnp.int32, (NCLS, CAP), 1))

    def b3_body(_, lh):
        lo, hi = lh
        mid = (lo + hi + 1) // 2
        c = jnp.sum(jnp.where(boundary & (flat <= mid), one, zero))
        le = c <= q
        return jnp.where(le, mid, lo), jnp.where(le, hi, mid - 1)

    flo, _ = jax.lax.fori_loop(
        0, 12, b3_body, (jnp.int32(-1), jnp.int32(NCLS * CAP - 1)))
    selected = above | (boundary & (flat <= flo))
    sel_cnt = jnp.sum(jnp.where(selected, one, zero))
    ext_min = jnp.min(jnp.where(selected, cs, inf))
    sel_ok = sel_cnt == jnp.minimum(total_kept, jnp.float32(KEEP))

    cap_ok = jnp.min(jnp.where(cnt_c <= jnp.float32(CAP), one, zero)) > 0.5
    full = cnt_c == tot_c
    all_full = jnp.min(jnp.where(full, one, zero)) > 0.5
    complete = jnp.min(jnp.where(full | (tau <= ext_min), one, zero)) > 0.5
    exact = cap_ok & fp_ok & sel_ok & jnp.where(
        sel_cnt == jnp.float32(KEEP), complete, all_full)
    flag_ref[...] = jnp.broadcast_to(jnp.where(exact, zero, one), (1, 1, 1))

    scale = jnp.float32(OUT_SZ)
    rx1 = jnp.round(cx1 * scale)
    ry1 = jnp.round(cy1 * scale)
    rx2 = jnp.round(cx2 * scale)
    ry2 = jnp.round(cy2 * scale)
    yy = jax.lax.broadcasted_iota(
        jnp.int32, (OUT_SZ, OUT_SZ, CAP), 0).astype(jnp.float32)
    xx = jax.lax.broadcasted_iota(
        jnp.int32, (OUT_SZ, OUT_SZ, CAP), 1).astype(jnp.float32)
    self32 = jnp.where(selected, one, zero)
    for c in range(NCLS):
        cover = ((yy >= ry1[c:c + 1, :][:, None, :])
                 & (yy < ry2[c:c + 1, :][:, None, :])
                 & (xx >= rx1[c:c + 1, :][:, None, :])
                 & (xx < rx2[c:c + 1, :][:, None, :])
                 & (self32[c:c + 1, :][:, None, :] > 0.5))
        vals = jnp.where(cover, cs[c:c + 1, :][:, None, :], inf)
        acc = jnp.min(vals, axis=2)
        out_ref[0, c] = jnp.where(jnp.isfinite(acc), acc, zero)
    out_ref[0, NCLS] = jnp.zeros((OUT_SZ, OUT_SZ), jnp.float32)


@functools.partial(jax.jit, static_argnames=("interpret",))
def _run_fast(x, interpret=False):
    xt = jnp.transpose(x, (0, 2, 1))
    b = x.shape[0]
    out, flags = pl.pallas_call(
        _fast_kernel,
        grid=(b,),
        in_specs=[pl.BlockSpec((1, 33, NPRIORS), lambda i: (i, 0, 0))],
        out_specs=[
            pl.BlockSpec((1, NUM_CLASSES_K, OUT_SZ, OUT_SZ),
                         lambda i: (i, 0, 0, 0)),
            pl.BlockSpec((1, 1, 1), lambda i: (i, 0, 0)),
        ],
        out_shape=[
            jax.ShapeDtypeStruct((b, NUM_CLASSES_K, OUT_SZ, OUT_SZ),
                                 jnp.float32),
            jax.ShapeDtypeStruct((b, 1, 1), jnp.float32),
        ],
        interpret=interpret,
    )(xt)
    return jnp.transpose(out, (0, 2, 3, 1)), flags


def kernel(x):
    out_fast, flags = _run_fast(x)
    any_bad = jnp.any(flags > 0.5)

    def slow(xx):
        return jnp.where(flags.reshape(-1, 1, 1, 1) > 0.5, _run(xx), out_fast)

    return jax.lax.cond(any_bad, slow, lambda xx: out_fast, x)


# speedup vs baseline: 10.3145x; 1.6685x over previous
import functools

import jax
import jax.numpy as jnp
from jax.experimental import pallas as pl

NUM_CLASSES_K = 21
NCLS = 20
NPRIORS = 5000
OUT_SZ = 19
KEEP = 200
IOU_T = 0.45
NEG_INF = float("-inf")

CAP = 64
TGT = 48.0
JAC_IT = 8
CHUNK = 512
NPAD = 5120
_HI = jax.lax.Precision.HIGHEST


def _decode(xt):
    loc0 = xt[0:1]
    loc1 = xt[1:2]
    loc2 = xt[2:3]
    loc3 = xt[3:4]
    p0 = xt[25:26]
    p1 = xt[26:27]
    p2 = xt[27:28]
    p3 = xt[28:29]
    v0 = xt[29:30]
    v1 = xt[30:31]
    v2 = xt[31:32]
    v3 = xt[32:33]
    pw = p2 - p0
    ph = p3 - p1
    pcx = 0.5 * (p2 + p0)
    pcy = 0.5 * (p3 + p1)
    cx = loc0 * pw * v0 + pcx
    cy = loc1 * pw * v1 + pcy
    w = jnp.exp(loc2 * v2) * pw
    h = jnp.exp(loc3 * v3) * ph
    x1 = jnp.clip(cx - 0.5 * w, 0.0, 1.0)
    y1 = jnp.clip(cy - 0.5 * h, 0.0, 1.0)
    x2 = jnp.clip(cx + 0.5 * w, 0.0, 1.0)
    y2 = jnp.clip(cy + 0.5 * h, 0.0, 1.0)
    area = (x2 - x1) * (y2 - y1)
    return x1, y1, x2, y2, area


def _paint_step(score, row, xmin, ymin, xmax, ymax, ok, acc):
    yy = jax.lax.broadcasted_iota(
        jnp.int32, (OUT_SZ, OUT_SZ, NUM_CLASSES_K), 0).astype(jnp.float32)
    xx = jax.lax.broadcasted_iota(
        jnp.int32, (OUT_SZ, OUT_SZ, NUM_CLASSES_K), 1).astype(jnp.float32)
    ch = jax.lax.broadcasted_iota(jnp.int32, (OUT_SZ, OUT_SZ, NUM_CLASSES_K), 2)
    cell = (yy >= ymin) & (yy < ymax) & (xx >= xmin) & (xx < xmax) & (ch == row)
    val = jnp.where(cell & ok, score, jnp.float32(jnp.inf))
    return jnp.minimum(acc, val)


def _nms_kernel(x_ref, out_ref):
    xt = x_ref[0]
    x1, y1, x2, y2, area = _decode(xt)
    confs = xt[5:25]
    sw0 = jnp.where(confs >= 0.6, confs, NEG_INF)
    iota_n = jax.lax.broadcasted_iota(jnp.int32, (1, NPRIORS), 1)

    def nms_body(k, carry):
        sw, ks, kx1, ky1, kx2, ky2 = carry
        m = jnp.max(sw, axis=1, keepdims=True)
        ok = m > NEG_INF
        idx = jnp.min(jnp.where(sw == m, iota_n, jnp.int32(2**30)),
                      axis=1, keepdims=True)
        pick = (iota_n == idx)
        px1 = jnp.sum(jnp.where(pick, x1, 0.0), axis=1, keepdims=True)
        py1 = jnp.sum(jnp.where(pick, y1, 0.0), axis=1, keepdims=True)
        px2 = jnp.sum(jnp.where(pick, x2, 0.0), axis=1, keepdims=True)
        py2 = jnp.sum(jnp.where(pick, y2, 0.0), axis=1, keepdims=True)
        parea = (px2 - px1) * (py2 - py1)
        xx1 = jnp.maximum(px1, x1)
        yy1 = jnp.maximum(py1, y1)
        xx2 = jnp.minimum(px2, x2)
        yy2 = jnp.minimum(py2, y2)
        inter = jnp.maximum(0.0, xx2 - xx1) * jnp.maximum(0.0, yy2 - yy1)
        union = jnp.maximum(parea + area - inter, 1e-12)
        iou = inter / union
        supp = (iou > IOU_T) | pick
        sw = jnp.where(ok & supp, NEG_INF, sw)
        mk = jnp.where(ok, m, NEG_INF)
        col = (jax.lax.broadcasted_iota(jnp.int32, (NCLS, KEEP), 1) == k)
        ks = jnp.where(col, mk, ks)
        kx1 = jnp.where(col, px1, kx1)
        ky1 = jnp.where(col, py1, ky1)
        kx2 = jnp.where(col, px2, kx2)
        ky2 = jnp.where(col, py2, ky2)
        return sw, ks, kx1, ky1, kx2, ky2

    zed = jnp.zeros((NCLS, KEEP), jnp.float32)
    ks0 = jnp.full((NCLS, KEEP), NEG_INF, jnp.float32)
    _, ks, kx1, ky1, kx2, ky2 = jax.lax.fori_loop(
        0, KEEP, nms_body, (sw0, ks0, zed, zed, zed, zed))

    scale = jnp.float32(OUT_SZ)
    rx1 = jnp.round(kx1 * scale)
    ry1 = jnp.round(ky1 * scale)
    rx2 = jnp.round(kx2 * scale)
    ry2 = jnp.round(ky2 * scale)

    flat = (jax.lax.broadcasted_iota(jnp.int32, (NCLS, KEEP), 0) * 256
            + jax.lax.broadcasted_iota(jnp.int32, (NCLS, KEEP), 1))

    def ext_body(k, carry):
        es, acc = carry
        m = jnp.max(es)
        ok = m > NEG_INF
        fk = jnp.min(jnp.where(es == m, flat, jnp.int32(2**30)))
        pick = (flat == fk)
        bx1 = jnp.sum(jnp.where(pick, rx1, 0.0))
        by1 = jnp.sum(jnp.where(pick, ry1, 0.0))
        bx2 = jnp.sum(jnp.where(pick, rx2, 0.0))
        by2 = jnp.sum(jnp.where(pick, ry2, 0.0))
        row = fk // 256
        acc = _paint_step(m, row, bx1, by1, bx2, by2, ok, acc)
        es = jnp.where(pick, NEG_INF, es)
        return es, acc

    acc0 = jnp.full((OUT_SZ, OUT_SZ, NUM_CLASSES_K), jnp.inf, jnp.float32)
    _, acc = jax.lax.fori_loop(0, KEEP, ext_body, (ks, acc0))
    out_ref[0] = jnp.where(jnp.isfinite(acc), acc, 0.0)


@functools.partial(jax.jit, static_argnames=("interpret",))
def _run(x, interpret=False):
    xt = jnp.transpose(x, (0, 2, 1))
    b = x.shape[0]
    return pl.pallas_call(
        _nms_kernel,
        grid=(b,),
        in_specs=[pl.BlockSpec((1, 33, NPRIORS), lambda i: (i, 0, 0))],
        out_specs=pl.BlockSpec((1, OUT_SZ, OUT_SZ, NUM_CLASSES_K),
                               lambda i: (i, 0, 0, 0)),
        out_shape=jax.ShapeDtypeStruct((b, OUT_SZ, OUT_SZ, NUM_CLASSES_K),
                                       jnp.float32),
        interpret=interpret,
    )(xt)


def _s1_kernel(xt_ref, cf_ref, posx_ref, tct_ref, dx1_ref, dy1_ref,
               dx2_ref, dy2_ref):
    one = jnp.float32(1.0)
    zero = jnp.float32(0.0)
    cf = cf_ref[...]
    rows = cf.shape[0]

    tot_c = jnp.sum(jnp.where(cf >= 0.6, one, zero), axis=1, keepdims=True)

    def bis_body(_, lh):
        lo, hi = lh
        mid = 0.5 * (lo + hi)
        cnt = jnp.sum(jnp.where(cf >= mid, one, zero), axis=1, keepdims=True)
        ge = cnt >= TGT
        return jnp.where(ge, mid, lo), jnp.where(ge, hi, mid)

    lo0 = jnp.full((rows, 1), 0.6, jnp.float32)
    hi0 = jnp.full((rows, 1), 1.0, jnp.float32)
    tau, _ = jax.lax.fori_loop(0, 16, bis_body, (lo0, hi0))

    maskf = jnp.where(cf >= tau, one, zero)
    cnt_c = jnp.sum(maskf, axis=1, keepdims=True)

    m3 = maskf.reshape(rows, 25, 200)
    st200 = jnp.where(
        jax.lax.broadcasted_iota(jnp.int32, (200, 200), 0)
        < jax.lax.broadcasted_iota(jnp.int32, (200, 200), 1), one, zero)
    pwi = jax.lax.dot_general(m3, st200, (((2,), (0,)), ((), ())),
                              preferred_element_type=jnp.float32,
                              precision=_HI)
    tots = jnp.sum(m3, axis=2)
    st25 = jnp.where(
        jax.lax.broadcasted_iota(jnp.int32, (25, 25), 0)
        < jax.lax.broadcasted_iota(jnp.int32, (25, 25), 1), one, zero)
    offs = jax.lax.dot_general(tots, st25, (((1,), (0,)), ((), ())),
                               preferred_element_type=jnp.float32,
                               precision=_HI)
    pos = (pwi + offs[:, :, None]).reshape(rows, NPRIORS)
    posx_ref[...] = jnp.where(maskf > 0.5, pos, jnp.float32(-1.0))
    tct_ref[...] = jnp.concatenate([tau, cnt_c, tot_c], axis=1)

    xt = xt_ref[...]
    loc0 = xt[:, 0:1, :]
    loc1 = xt[:, 1:2, :]
    loc2 = xt[:, 2:3, :]
    loc3 = xt[:, 3:4, :]
    p0 = xt[:, 25:26, :]
    p1 = xt[:, 26:27, :]
    p2 = xt[:, 27:28, :]
    p3 = xt[:, 28:29, :]
    v0 = xt[:, 29:30, :]
    v1 = xt[:, 30:31, :]
    v2 = xt[:, 31:32, :]
    v3 = xt[:, 32:33, :]
    pw = p2 - p0
    ph = p3 - p1
    pcx = 0.5 * (p2 + p0)
    pcy = 0.5 * (p3 + p1)
    cx = loc0 * pw * v0 + pcx
    cy = loc1 * pw * v1 + pcy
    w = jnp.exp(loc2 * v2) * pw
    h = jnp.exp(loc3 * v3) * ph
    dx1_ref[...] = jnp.clip(cx - 0.5 * w, 0.0, 1.0)[:, 0, :]
    dy1_ref[...] = jnp.clip(cy - 0.5 * h, 0.0, 1.0)[:, 0, :]
    dx2_ref[...] = jnp.clip(cx + 0.5 * w, 0.0, 1.0)[:, 0, :]
    dy2_ref[...] = jnp.clip(cy + 0.5 * h, 0.0, 1.0)[:, 0, :]


def _s2_kernel(posx_ref, cf_ref, x1_ref, y1_ref, x2_ref, y2_ref, comp_ref):
    one = jnp.float32(1.0)
    zero = jnp.float32(0.0)

    @pl.when(pl.program_id(1) == 0)
    def _():
        comp_ref[...] = jnp.zeros_like(comp_ref)

    posm = posx_ref[...][:, None, :]
    kio = jax.lax.broadcasted_iota(jnp.int32, (1, CAP, 1), 1).astype(jnp.float32)
    ohc = jnp.where(posm == kio, one, zero)
    vals = jnp.concatenate([
        cf_ref[...][:, :, None],
        x1_ref[...][:, :, None],
        y1_ref[...][:, :, None],
        x2_ref[...][:, :, None],
        y2_ref[...][:, :, None],
    ], axis=2)
    comp_ref[...] += jax.lax.dot_general(
        ohc, vals, (((2,), (1,)), ((0,), (0,))),
        preferred_element_type=jnp.float32, precision=_HI)


def _s3_kernel(comp_ref, tct_ref, cs_ref, r1x_ref, r1y_ref, r2x_ref,
               r2y_ref, sel_ref, flag_ref):
    one = jnp.float32(1.0)
    zero = jnp.float32(0.0)
    inf = jnp.float32(jnp.inf)
    comp = comp_ref[...]
    rows = comp.shape[0]
    b = rows // NCLS
    tct = tct_ref[...]
    tau = tct[:, 0:1]
    cnt_c = tct[:, 1:2]
    tot_c = tct[:, 2:3]

    cs = comp[:, :, 0]
    cx1 = comp[:, :, 1]
    cy1 = comp[:, :, 2]
    cx2 = comp[:, :, 3]
    cy2 = comp[:, :, 4]

    slot = jax.lax.broadcasted_iota(jnp.int32, (rows, CAP), 1).astype(jnp.float32)
    validf = jnp.where(slot < cnt_c, one, zero)

    ai = (cx2 - cx1) * (cy2 - cy1)
    xx1 = jnp.maximum(cx1[:, :, None], cx1[:, None, :])
    yy1 = jnp.maximum(cy1[:, :, None], cy1[:, None, :])
    xx2 = jnp.minimum(cx2[:, :, None], cx2[:, None, :])
    yy2 = jnp.minimum(cy2[:, :, None], cy2[:, None, :])
    inter = jnp.maximum(zero, xx2 - xx1) * jnp.maximum(zero, yy2 - yy1)
    union = jnp.maximum(ai[:, :, None] + ai[:, None, :] - inter, 1e-12)
    iou = inter / union
    si = jax.lax.broadcasted_iota(jnp.int32, (1, CAP, CAP), 1)
    sj = jax.lax.broadcasted_iota(jnp.int32, (1, CAP, CAP), 2)
    higher = ((cs[:, :, None] > cs[:, None, :])
              | ((cs[:, :, None] == cs[:, None, :]) & (si < sj)))
    vpair = (validf[:, :, None] > 0.5) & (validf[:, None, :] > 0.5)
    sm = jnp.where(higher & (iou > IOU_T) & vpair, one, zero)

    def jac(k):
        sup = jax.lax.dot_general(sm, k, (((1,), (1,)), ((0,), (0,))),
                                  preferred_element_type=jnp.float32,
                                  precision=_HI)
        return jnp.where((sup < 0.5) & (validf > 0.5), one, zero)

    kcur = jax.lax.fori_loop(0, JAC_IT, lambda _, k: jac(k), validf)
    knext = jac(kcur)
    fp_ok = jnp.min(jnp.where(knext == kcur, one, zero).reshape(b, NCLS, CAP),
                    axis=(1, 2), keepdims=True)

    cs3 = cs.reshape(b, NCLS, CAP)
    kp3 = knext.reshape(b, NCLS, CAP)
    keep3 = kp3 > 0.5
    total_kept = jnp.sum(kp3, axis=(1, 2), keepdims=True)

    def b2_body(_, lh):
        lo, hi = lh
        mid = 0.5 * (lo + hi)
        cnt = jnp.sum(jnp.where(keep3 & (cs3 >= mid), one, zero),
                      axis=(1, 2), keepdims=True)
        ge = cnt >= jnp.float32(KEEP)
        return jnp.where(ge, mid, lo), jnp.where(ge, hi, mid)

    glo0 = jnp.full((b, 1, 1), 0.5, jnp.float32)
    ghi0 = jnp.full((b, 1, 1), 1.0, jnp.float32)
    glo, _ = jax.lax.fori_loop(0, 26, b2_body, (glo0, ghi0))

    tsel = keep3 & (cs3 >= glo)
    bval = jnp.min(jnp.where(tsel, cs3, inf), axis=(1, 2), keepdims=True)
    above = tsel & (cs3 > bval)
    na = jnp.sum(jnp.where(above, one, zero), axis=(1, 2), keepdims=True)
    boundary = tsel & (cs3 == bval)
    q = jnp.float32(KEEP) - na
    flat3 = (jax.lax.broadcasted_iota(jnp.int32, (1, NCLS, CAP), 1) * CAP
             + jax.lax.broadcasted_iota(jnp.int32, (1, NCLS, CAP), 2))

    def b3_body(_, lh):
        lo, hi = lh
        mid = (lo + hi + 1) // 2
        c = jnp.sum(jnp.where(boundary & (flat3 <= mid), one, zero),
                    axis=(1, 2), keepdims=True)
        le = c <= q
        return jnp.where(le, mid, lo), jnp.where(le, hi, mid - 1)

    flo0 = jnp.full((b, 1, 1), -1, jnp.int32)
    fhi0 = jnp.full((b, 1, 1), NCLS * CAP - 1, jnp.int32)
    flo, _ = jax.lax.fori_loop(0, 12, b3_body, (flo0, fhi0))

    selected = above | (boundary & (flat3 <= flo))
    self32 = jnp.where(selected, one, zero)
    sel_cnt = jnp.sum(self32, axis=(1, 2), keepdims=True)
    ext_min = jnp.min(jnp.where(selected, cs3, inf), axis=(1, 2),
                      keepdims=True)
    sel_ok = jnp.where(
        sel_cnt == jnp.minimum(total_kept, jnp.float32(KEEP)), one, zero)

    cnt2 = cnt_c.reshape(b, NCLS)
    tot2 = tot_c.reshape(b, NCLS)
    tau2 = tau.reshape(b, NCLS)
    cap_ok = jnp.min(jnp.where(cnt2 <= jnp.float32(CAP), one, zero),
                     axis=1, keepdims=True)[:, :, None]
    fullf = jnp.where(cnt2 == tot2, one, zero)
    all_full = jnp.min(fullf, axis=1, keepdims=True)[:, :, None]
    emin2 = ext_min.reshape(b, 1)
    complete = jnp.min(
        jnp.where((fullf > 0.5) | (tau2 <= emin2), one, zero),
        axis=1, keepdims=True)[:, :, None]
    exact = (cap_ok * fp_ok * sel_ok
             * jnp.where(sel_cnt == jnp.float32(KEEP), complete, all_full))
    flag_ref[...] = one - jnp.minimum(exact, one)

    scale = jnp.float32(OUT_SZ)
    cs_ref[...] = cs3
    r1x_ref[...] = jnp.round(cx1 * scale).reshape(b, NCLS, CAP)
    r1y_ref[...] = jnp.round(cy1 * scale).reshape(b, NCLS, CAP)
    r2x_ref[...] = jnp.round(cx2 * scale).reshape(b, NCLS, CAP)
    r2y_ref[...] = jnp.round(cy2 * scale).reshape(b, NCLS, CAP)
    sel_ref[...] = self32


def _s4_kernel(cs_ref, r1x_ref, r1y_ref, r2x_ref, r2y_ref, sel_ref, out_ref):
    zero = jnp.float32(0.0)
    inf = jnp.float32(jnp.inf)
    yy = jax.lax.broadcasted_iota(
        jnp.int32, (1, OUT_SZ, OUT_SZ, CAP), 1).astype(jnp.float32)
    xx = jax.lax.broadcasted_iota(
        jnp.int32, (1, OUT_SZ, OUT_SZ, CAP), 2).astype(jnp.float32)
    ry1c = r1y_ref[...]
    ry2c = r2y_ref[...]
    rx1c = r1x_ref[...]
    rx2c = r2x_ref[...]
    selc = sel_ref[...]
    csc = cs_ref[...]
    cover = ((yy >= ry1c) & (yy < ry2c) & (xx >= rx1c) & (xx < rx2c)
             & (selc > 0.5))
    vals = jnp.where(cover, csc, inf)
    acc = jnp.min(vals, axis=3)
    out_ref[...] = jnp.where(jnp.isfinite(acc), acc, zero)[:, None, :, :]


@functools.partial(jax.jit, static_argnames=("interpret",))
def _run_fast(x, interpret=False):
    b = x.shape[0]
    rows = b * NCLS
    xt = jnp.transpose(x, (0, 2, 1))
    cfr = jnp.transpose(x[:, :, 5:25], (0, 2, 1)).reshape(rows, NPRIORS)

    f32 = jnp.float32
    posx, tct, dx1, dy1, dx2, dy2 = pl.pallas_call(
        _s1_kernel,
        out_shape=[
            jax.ShapeDtypeStruct((rows, NPRIORS), f32),
            jax.ShapeDtypeStruct((rows, 3), f32),
            jax.ShapeDtypeStruct((b, NPRIORS), f32),
            jax.ShapeDtypeStruct((b, NPRIORS), f32),
            jax.ShapeDtypeStruct((b, NPRIORS), f32),
            jax.ShapeDtypeStruct((b, NPRIORS), f32),
        ],
        interpret=interpret,
    )(xt, cfr)

    x1r = jnp.broadcast_to(dx1[:, None, :], (b, NCLS, NPRIORS)).reshape(
        rows, NPRIORS)
    y1r = jnp.broadcast_to(dy1[:, None, :], (b, NCLS, NPRIORS)).reshape(
        rows, NPRIORS)
    x2r = jnp.broadcast_to(dx2[:, None, :], (b, NCLS, NPRIORS)).reshape(
        rows, NPRIORS)
    y2r = jnp.broadcast_to(dy2[:, None, :], (b, NCLS, NPRIORS)).reshape(
        rows, NPRIORS)

    pad = ((0, 0), (0, NPAD - NPRIORS))
    posx_p = jnp.pad(posx, pad, constant_values=-1.0)
    cfr_p = jnp.pad(cfr, pad)
    x1r_p = jnp.pad(x1r, pad)
    y1r_p = jnp.pad(y1r, pad)
    x2r_p = jnp.pad(x2r, pad)
    y2r_p = jnp.pad(y2r, pad)

    nchunks = NPAD // CHUNK
    rgrp = 40
    chunk_spec = pl.BlockSpec((rgrp, CHUNK), lambda j, i: (j, i))
    comp = pl.pallas_call(
        _s2_kernel,
        grid=(rows // rgrp, nchunks),
        in_specs=[chunk_spec] * 6,
        out_specs=pl.BlockSpec((rgrp, CAP, 5), lambda j, i: (j, 0, 0)),
        out_shape=jax.ShapeDtypeStruct((rows, CAP, 5), f32),
        interpret=interpret,
    )(posx_p, cfr_p, x1r_p, y1r_p, x2r_p, y2r_p)

    cs3, r1x, r1y, r2x, r2y, sel, flags = pl.pallas_call(
        _s3_kernel,
        out_shape=[jax.ShapeDtypeStruct((b, NCLS, CAP), f32)] * 6
        + [jax.ShapeDtypeStruct((b, 1, 1), f32)],
        interpret=interpret,
    )(comp, tct)

    cs4, r1x4, r1y4, r2x4, r2y4, sel4 = (
        a.reshape(b, NCLS, 1, CAP) for a in (cs3, r1x, r1y, r2x, r2y, sel))
    cls_spec = pl.BlockSpec((b, 1, 1, CAP), lambda i: (0, i, 0, 0))
    painted = pl.pallas_call(
        _s4_kernel,
        grid=(NCLS,),
        in_specs=[cls_spec] * 6,
        out_specs=pl.BlockSpec((b, 1, OUT_SZ, OUT_SZ), lambda i: (0, i, 0, 0)),
        out_shape=jax.ShapeDtypeStruct((b, NCLS, OUT_SZ, OUT_SZ), f32),
        interpret=interpret,
    )(cs4, r1x4, r1y4, r2x4, r2y4, sel4)

    out = jnp.concatenate(
        [jnp.transpose(painted, (0, 2, 3, 1)),
         jnp.zeros((b, OUT_SZ, OUT_SZ, 1), f32)], axis=3)
    return out, flags


def kernel(x):
    out_fast, flags = _run_fast(x)
    any_bad = jnp.any(flags > 0.5)

    def slow(xx):
        return jnp.where(flags.reshape(-1, 1, 1, 1) > 0.5, _run(xx), out_fast)

    return jax.lax.cond(any_bad, slow, lambda xx: out_fast, x)


# speedup vs baseline: 18.3675x; 1.7808x over previous
import functools

import jax
import jax.numpy as jnp
from jax.experimental import pallas as pl

NUM_CLASSES_K = 21
NCLS = 20
NPRIORS = 5000
OUT_SZ = 19
KEEP = 200
IOU_T = 0.45
NEG_INF = float("-inf")

CAP = 64
TGT = 48.0
JAC_IT = 8
CHUNK = 512
NPAD = 5120
_HI = jax.lax.Precision.HIGHEST


def _decode(xt):
    loc0 = xt[0:1]
    loc1 = xt[1:2]
    loc2 = xt[2:3]
    loc3 = xt[3:4]
    p0 = xt[25:26]
    p1 = xt[26:27]
    p2 = xt[27:28]
    p3 = xt[28:29]
    v0 = xt[29:30]
    v1 = xt[30:31]
    v2 = xt[31:32]
    v3 = xt[32:33]
    pw = p2 - p0
    ph = p3 - p1
    pcx = 0.5 * (p2 + p0)
    pcy = 0.5 * (p3 + p1)
    cx = loc0 * pw * v0 + pcx
    cy = loc1 * pw * v1 + pcy
    w = jnp.exp(loc2 * v2) * pw
    h = jnp.exp(loc3 * v3) * ph
    x1 = jnp.clip(cx - 0.5 * w, 0.0, 1.0)
    y1 = jnp.clip(cy - 0.5 * h, 0.0, 1.0)
    x2 = jnp.clip(cx + 0.5 * w, 0.0, 1.0)
    y2 = jnp.clip(cy + 0.5 * h, 0.0, 1.0)
    area = (x2 - x1) * (y2 - y1)
    return x1, y1, x2, y2, area


def _paint_step(score, row, xmin, ymin, xmax, ymax, ok, acc):
    yy = jax.lax.broadcasted_iota(
        jnp.int32, (OUT_SZ, OUT_SZ, NUM_CLASSES_K), 0).astype(jnp.float32)
    xx = jax.lax.broadcasted_iota(
        jnp.int32, (OUT_SZ, OUT_SZ, NUM_CLASSES_K), 1).astype(jnp.float32)
    ch = jax.lax.broadcasted_iota(jnp.int32, (OUT_SZ, OUT_SZ, NUM_CLASSES_K), 2)
    cell = (yy >= ymin) & (yy < ymax) & (xx >= xmin) & (xx < xmax) & (ch == row)
    val = jnp.where(cell & ok, score, jnp.float32(jnp.inf))
    return jnp.minimum(acc, val)


def _nms_kernel(x_ref, out_ref):
    xt = x_ref[0]
    x1, y1, x2, y2, area = _decode(xt)
    confs = xt[5:25]
    sw0 = jnp.where(confs >= 0.6, confs, NEG_INF)
    iota_n = jax.lax.broadcasted_iota(jnp.int32, (1, NPRIORS), 1)

    def nms_body(k, carry):
        sw, ks, kx1, ky1, kx2, ky2 = carry
        m = jnp.max(sw, axis=1, keepdims=True)
        ok = m > NEG_INF
        idx = jnp.min(jnp.where(sw == m, iota_n, jnp.int32(2**30)),
                      axis=1, keepdims=True)
        pick = (iota_n == idx)
        px1 = jnp.sum(jnp.where(pick, x1, 0.0), axis=1, keepdims=True)
        py1 = jnp.sum(jnp.where(pick, y1, 0.0), axis=1, keepdims=True)
        px2 = jnp.sum(jnp.where(pick, x2, 0.0), axis=1, keepdims=True)
        py2 = jnp.sum(jnp.where(pick, y2, 0.0), axis=1, keepdims=True)
        parea = (px2 - px1) * (py2 - py1)
        xx1 = jnp.maximum(px1, x1)
        yy1 = jnp.maximum(py1, y1)
        xx2 = jnp.minimum(px2, x2)
        yy2 = jnp.minimum(py2, y2)
        inter = jnp.maximum(0.0, xx2 - xx1) * jnp.maximum(0.0, yy2 - yy1)
        union = jnp.maximum(parea + area - inter, 1e-12)
        iou = inter / union
        supp = (iou > IOU_T) | pick
        sw = jnp.where(ok & supp, NEG_INF, sw)
        mk = jnp.where(ok, m, NEG_INF)
        col = (jax.lax.broadcasted_iota(jnp.int32, (NCLS, KEEP), 1) == k)
        ks = jnp.where(col, mk, ks)
        kx1 = jnp.where(col, px1, kx1)
        ky1 = jnp.where(col, py1, ky1)
        kx2 = jnp.where(col, px2, kx2)
        ky2 = jnp.where(col, py2, ky2)
        return sw, ks, kx1, ky1, kx2, ky2

    zed = jnp.zeros((NCLS, KEEP), jnp.float32)
    ks0 = jnp.full((NCLS, KEEP), NEG_INF, jnp.float32)
    _, ks, kx1, ky1, kx2, ky2 = jax.lax.fori_loop(
        0, KEEP, nms_body, (sw0, ks0, zed, zed, zed, zed))

    scale = jnp.float32(OUT_SZ)
    rx1 = jnp.round(kx1 * scale)
    ry1 = jnp.round(ky1 * scale)
    rx2 = jnp.round(kx2 * scale)
    ry2 = jnp.round(ky2 * scale)

    flat = (jax.lax.broadcasted_iota(jnp.int32, (NCLS, KEEP), 0) * 256
            + jax.lax.broadcasted_iota(jnp.int32, (NCLS, KEEP), 1))

    def ext_body(k, carry):
        es, acc = carry
        m = jnp.max(es)
        ok = m > NEG_INF
        fk = jnp.min(jnp.where(es == m, flat, jnp.int32(2**30)))
        pick = (flat == fk)
        bx1 = jnp.sum(jnp.where(pick, rx1, 0.0))
        by1 = jnp.sum(jnp.where(pick, ry1, 0.0))
        bx2 = jnp.sum(jnp.where(pick, rx2, 0.0))
        by2 = jnp.sum(jnp.where(pick, ry2, 0.0))
        row = fk // 256
        acc = _paint_step(m, row, bx1, by1, bx2, by2, ok, acc)
        es = jnp.where(pick, NEG_INF, es)
        return es, acc

    acc0 = jnp.full((OUT_SZ, OUT_SZ, NUM_CLASSES_K), jnp.inf, jnp.float32)
    _, acc = jax.lax.fori_loop(0, KEEP, ext_body, (ks, acc0))
    out_ref[0] = jnp.where(jnp.isfinite(acc), acc, 0.0)


@functools.partial(jax.jit, static_argnames=("interpret",))
def _run(x, interpret=False):
    xt = jnp.transpose(x, (0, 2, 1))
    b = x.shape[0]
    return pl.pallas_call(
        _nms_kernel,
        grid=(b,),
        in_specs=[pl.BlockSpec((1, 33, NPRIORS), lambda i: (i, 0, 0))],
        out_specs=pl.BlockSpec((1, OUT_SZ, OUT_SZ, NUM_CLASSES_K),
                               lambda i: (i, 0, 0, 0)),
        out_shape=jax.ShapeDtypeStruct((b, OUT_SZ, OUT_SZ, NUM_CLASSES_K),
                                       jnp.float32),
        interpret=interpret,
    )(xt)


def _s1_kernel(xt_ref, cfp_ref, posx_ref, tct_ref, dx1_ref, dy1_ref,
               dx2_ref, dy2_ref):
    one = jnp.float32(1.0)
    zero = jnp.float32(0.0)
    xt = xt_ref[...]
    b = xt.shape[0]
    cf3 = xt[:, 5:25, :]
    cf = cf3.reshape(b * NCLS, NPRIORS)
    rows = cf.shape[0]

    tot_c = jnp.sum(jnp.where(cf >= 0.6, one, zero), axis=1, keepdims=True)

    def bis_body(_, lh):
        lo, hi = lh
        mid = 0.5 * (lo + hi)
        cnt = jnp.sum(jnp.where(cf >= mid, one, zero), axis=1, keepdims=True)
        ge = cnt >= TGT
        return jnp.where(ge, mid, lo), jnp.where(ge, hi, mid)

    lo0 = jnp.full((rows, 1), 0.6, jnp.float32)
    hi0 = jnp.full((rows, 1), 1.0, jnp.float32)
    tau, _ = jax.lax.fori_loop(0, 16, bis_body, (lo0, hi0))

    maskf = jnp.where(cf >= tau, one, zero)
    cnt_c = jnp.sum(maskf, axis=1, keepdims=True)

    m3 = maskf.reshape(rows, 25, 200)
    st200 = jnp.where(
        jax.lax.broadcasted_iota(jnp.int32, (200, 200), 0)
        < jax.lax.broadcasted_iota(jnp.int32, (200, 200), 1), one, zero)
    pwi = jax.lax.dot_general(m3, st200, (((2,), (0,)), ((), ())),
                              preferred_element_type=jnp.float32,
                              precision=_HI)
    tots = jnp.sum(m3, axis=2)
    st25 = jnp.where(
        jax.lax.broadcasted_iota(jnp.int32, (25, 25), 0)
        < jax.lax.broadcasted_iota(jnp.int32, (25, 25), 1), one, zero)
    offs = jax.lax.dot_general(tots, st25, (((1,), (0,)), ((), ())),
                               preferred_element_type=jnp.float32,
                               precision=_HI)
    pos = (pwi + offs[:, :, None]).reshape(rows, NPRIORS)
    posx = jnp.where(maskf > 0.5, pos, jnp.float32(-1.0))
    posx_ref[:, :, :NPRIORS] = posx.reshape(b, NCLS, NPRIORS)
    posx_ref[:, :, NPRIORS:] = jnp.full((b, NCLS, NPAD - NPRIORS), -1.0,
                                        jnp.float32)
    cfp_ref[:, :, :NPRIORS] = cf3
    cfp_ref[:, :, NPRIORS:] = jnp.zeros((b, NCLS, NPAD - NPRIORS), jnp.float32)
    tct_ref[...] = jnp.concatenate([tau, cnt_c, tot_c], axis=1)

    loc0 = xt[:, 0:1, :]
    loc1 = xt[:, 1:2, :]
    loc2 = xt[:, 2:3, :]
    loc3 = xt[:, 3:4, :]
    p0 = xt[:, 25:26, :]
    p1 = xt[:, 26:27, :]
    p2 = xt[:, 27:28, :]
    p3 = xt[:, 28:29, :]
    v0 = xt[:, 29:30, :]
    v1 = xt[:, 30:31, :]
    v2 = xt[:, 31:32, :]
    v3 = xt[:, 32:33, :]
    pw = p2 - p0
    ph = p3 - p1
    pcx = 0.5 * (p2 + p0)
    pcy = 0.5 * (p3 + p1)
    cx = loc0 * pw * v0 + pcx
    cy = loc1 * pw * v1 + pcy
    w = jnp.exp(loc2 * v2) * pw
    h = jnp.exp(loc3 * v3) * ph
    zpad = jnp.zeros((b, 1, NPAD - NPRIORS), jnp.float32)
    dx1_ref[:, :, :NPRIORS] = jnp.clip(cx - 0.5 * w, 0.0, 1.0)
    dx1_ref[:, :, NPRIORS:] = zpad
    dy1_ref[:, :, :NPRIORS] = jnp.clip(cy - 0.5 * h, 0.0, 1.0)
    dy1_ref[:, :, NPRIORS:] = zpad
    dx2_ref[:, :, :NPRIORS] = jnp.clip(cx + 0.5 * w, 0.0, 1.0)
    dx2_ref[:, :, NPRIORS:] = zpad
    dy2_ref[:, :, :NPRIORS] = jnp.clip(cy + 0.5 * h, 0.0, 1.0)
    dy2_ref[:, :, NPRIORS:] = zpad


def _s2_kernel(posx_ref, cf_ref, x1_ref, y1_ref, x2_ref, y2_ref, comp_ref):
    one = jnp.float32(1.0)
    zero = jnp.float32(0.0)

    @pl.when(pl.program_id(1) == 0)
    def _():
        comp_ref[...] = jnp.zeros_like(comp_ref)

    posm = posx_ref[0][:, None, :]
    kio = jax.lax.broadcasted_iota(jnp.int32, (1, CAP, 1), 1).astype(jnp.float32)
    ohc = jnp.where(posm == kio, one, zero)
    vals = jnp.concatenate([
        cf_ref[0][:, :, None],
        jnp.broadcast_to(x1_ref[0], (NCLS, CHUNK))[:, :, None],
        jnp.broadcast_to(y1_ref[0], (NCLS, CHUNK))[:, :, None],
        jnp.broadcast_to(x2_ref[0], (NCLS, CHUNK))[:, :, None],
        jnp.broadcast_to(y2_ref[0], (NCLS, CHUNK))[:, :, None],
    ], axis=2)
    comp_ref[...] += jax.lax.dot_general(
        ohc, vals, (((2,), (1,)), ((0,), (0,))),
        preferred_element_type=jnp.float32, precision=_HI)


def _s3_kernel(comp_ref, tct_ref, cs_ref, r1x_ref, r1y_ref, r2x_ref,
               r2y_ref, sel_ref, flag_ref):
    one = jnp.float32(1.0)
    zero = jnp.float32(0.0)
    inf = jnp.float32(jnp.inf)
    comp = comp_ref[...]
    rows = comp.shape[0]
    b = rows // NCLS
    tct = tct_ref[...]
    tau = tct[:, 0:1]
    cnt_c = tct[:, 1:2]
    tot_c = tct[:, 2:3]

    cs = comp[:, :, 0]
    cx1 = comp[:, :, 1]
    cy1 = comp[:, :, 2]
    cx2 = comp[:, :, 3]
    cy2 = comp[:, :, 4]

    slot = jax.lax.broadcasted_iota(jnp.int32, (rows, CAP), 1).astype(jnp.float32)
    validf = jnp.where(slot < cnt_c, one, zero)

    ai = (cx2 - cx1) * (cy2 - cy1)
    xx1 = jnp.maximum(cx1[:, :, None], cx1[:, None, :])
    yy1 = jnp.maximum(cy1[:, :, None], cy1[:, None, :])
    xx2 = jnp.minimum(cx2[:, :, None], cx2[:, None, :])
    yy2 = jnp.minimum(cy2[:, :, None], cy2[:, None, :])
    inter = jnp.maximum(zero, xx2 - xx1) * jnp.maximum(zero, yy2 - yy1)
    union = jnp.maximum(ai[:, :, None] + ai[:, None, :] - inter, 1e-12)
    iou = inter / union
    si = jax.lax.broadcasted_iota(jnp.int32, (1, CAP, CAP), 1)
    sj = jax.lax.broadcasted_iota(jnp.int32, (1, CAP, CAP), 2)
    higher = ((cs[:, :, None] > cs[:, None, :])
              | ((cs[:, :, None] == cs[:, None, :]) & (si < sj)))
    vpair = (validf[:, :, None] > 0.5) & (validf[:, None, :] > 0.5)
    sm = jnp.where(higher & (iou > IOU_T) & vpair, one, zero)

    def jac(k):
        sup = jax.lax.dot_general(sm, k, (((1,), (1,)), ((0,), (0,))),
                                  preferred_element_type=jnp.float32,
                                  precision=_HI)
        return jnp.where((sup < 0.5) & (validf > 0.5), one, zero)

    kcur = jax.lax.fori_loop(0, JAC_IT, lambda _, k: jac(k), validf)
    knext = jac(kcur)
    fp_ok = jnp.min(jnp.where(knext == kcur, one, zero).reshape(b, NCLS, CAP),
                    axis=(1, 2), keepdims=True)

    cs3 = cs.reshape(b, NCLS, CAP)
    kp3 = knext.reshape(b, NCLS, CAP)
    keep3 = kp3 > 0.5
    total_kept = jnp.sum(kp3, axis=(1, 2), keepdims=True)

    def b2_body(_, lh):
        lo, hi = lh
        mid = 0.5 * (lo + hi)
        cnt = jnp.sum(jnp.where(keep3 & (cs3 >= mid), one, zero),
                      axis=(1, 2), keepdims=True)
        ge = cnt >= jnp.float32(KEEP)
        return jnp.where(ge, mid, lo), jnp.where(ge, hi, mid)

    glo0 = jnp.full((b, 1, 1), 0.5, jnp.float32)
    ghi0 = jnp.full((b, 1, 1), 1.0, jnp.float32)
    glo, _ = jax.lax.fori_loop(0, 26, b2_body, (glo0, ghi0))

    tsel = keep3 & (cs3 >= glo)
    bval = jnp.min(jnp.where(tsel, cs3, inf), axis=(1, 2), keepdims=True)
    above = tsel & (cs3 > bval)
    na = jnp.sum(jnp.where(above, one, zero), axis=(1, 2), keepdims=True)
    boundary = tsel & (cs3 == bval)
    q = jnp.float32(KEEP) - na
    flat3 = (jax.lax.broadcasted_iota(jnp.int32, (1, NCLS, CAP), 1) * CAP
             + jax.lax.broadcasted_iota(jnp.int32, (1, NCLS, CAP), 2))

    def b3_body(_, lh):
        lo, hi = lh
        mid = (lo + hi + 1) // 2
        c = jnp.sum(jnp.where(boundary & (flat3 <= mid), one, zero),
                    axis=(1, 2), keepdims=True)
        le = c <= q
        return jnp.where(le, mid, lo), jnp.where(le, hi, mid - 1)

    flo0 = jnp.full((b, 1, 1), -1, jnp.int32)
    fhi0 = jnp.full((b, 1, 1), NCLS * CAP - 1, jnp.int32)
    flo, _ = jax.lax.fori_loop(0, 12, b3_body, (flo0, fhi0))

    selected = above | (boundary & (flat3 <= flo))
    self32 = jnp.where(selected, one, zero)
    sel_cnt = jnp.sum(self32, axis=(1, 2), keepdims=True)
    ext_min = jnp.min(jnp.where(selected, cs3, inf), axis=(1, 2),
                      keepdims=True)
    sel_ok = jnp.where(
        sel_cnt == jnp.minimum(total_kept, jnp.float32(KEEP)), one, zero)

    cnt2 = cnt_c.reshape(b, NCLS)
    tot2 = tot_c.reshape(b, NCLS)
    tau2 = tau.reshape(b, NCLS)
    cap_ok = jnp.min(jnp.where(cnt2 <= jnp.float32(CAP), one, zero),
                     axis=1, keepdims=True)[:, :, None]
    fullf = jnp.where(cnt2 == tot2, one, zero)
    all_full = jnp.min(fullf, axis=1, keepdims=True)[:, :, None]
    emin2 = ext_min.reshape(b, 1)
    complete = jnp.min(
        jnp.where((fullf > 0.5) | (tau2 <= emin2), one, zero),
        axis=1, keepdims=True)[:, :, None]
    exact = (cap_ok * fp_ok * sel_ok
             * jnp.where(sel_cnt == jnp.float32(KEEP), complete, all_full))
    flag_ref[...] = one - jnp.minimum(exact, one)

    scale = jnp.float32(OUT_SZ)
    cs_ref[...] = cs3
    r1x_ref[...] = jnp.round(cx1 * scale).reshape(b, NCLS, CAP)
    r1y_ref[...] = jnp.round(cy1 * scale).reshape(b, NCLS, CAP)
    r2x_ref[...] = jnp.round(cx2 * scale).reshape(b, NCLS, CAP)
    r2y_ref[...] = jnp.round(cy2 * scale).reshape(b, NCLS, CAP)
    sel_ref[...] = self32


def _s4_kernel(cs_ref, r1x_ref, r1y_ref, r2x_ref, r2y_ref, sel_ref, out_ref):
    zero = jnp.float32(0.0)
    inf = jnp.float32(jnp.inf)
    yy = jax.lax.broadcasted_iota(
        jnp.int32, (1, OUT_SZ, OUT_SZ, CAP), 1).astype(jnp.float32)
    xx = jax.lax.broadcasted_iota(
        jnp.int32, (1, OUT_SZ, OUT_SZ, CAP), 2).astype(jnp.float32)
    ry1c = r1y_ref[...]
    ry2c = r2y_ref[...]
    rx1c = r1x_ref[...]
    rx2c = r2x_ref[...]
    selc = sel_ref[...]
    csc = cs_ref[...]
    cover = ((yy >= ry1c) & (yy < ry2c) & (xx >= rx1c) & (xx < rx2c)
             & (selc > 0.5))
    vals = jnp.where(cover, csc, inf)
    acc = jnp.min(vals, axis=3)
    out_ref[...] = jnp.where(jnp.isfinite(acc), acc, zero)[:, None, :, :]


@functools.partial(jax.jit, static_argnames=("interpret",))
def _run_fast(x, interpret=False):
    b = x.shape[0]
    rows = b * NCLS
    xt = jnp.transpose(x, (0, 2, 1))

    f32 = jnp.float32
    cfp, posx, tct, dx1, dy1, dx2, dy2 = pl.pallas_call(
        _s1_kernel,
        out_shape=[
            jax.ShapeDtypeStruct((b, NCLS, NPAD), f32),
            jax.ShapeDtypeStruct((b, NCLS, NPAD), f32),
            jax.ShapeDtypeStruct((rows, 3), f32),
            jax.ShapeDtypeStruct((b, 1, NPAD), f32),
            jax.ShapeDtypeStruct((b, 1, NPAD), f32),
            jax.ShapeDtypeStruct((b, 1, NPAD), f32),
            jax.ShapeDtypeStruct((b, 1, NPAD), f32),
        ],
        interpret=interpret,
    )(xt)

    nchunks = NPAD // CHUNK
    row_spec = pl.BlockSpec((1, NCLS, CHUNK), lambda j, i: (j, 0, i))
    box_spec = pl.BlockSpec((1, 1, CHUNK), lambda j, i: (j, 0, i))
    comp = pl.pallas_call(
        _s2_kernel,
        grid=(b, nchunks),
        in_specs=[row_spec, row_spec, box_spec, box_spec, box_spec, box_spec],
        out_specs=pl.BlockSpec((NCLS, CAP, 5), lambda j, i: (j, 0, 0)),
        out_shape=jax.ShapeDtypeStruct((rows, CAP, 5), f32),
        interpret=interpret,
    )(posx, cfp, dx1, dy1, dx2, dy2)

    cs3, r1x, r1y, r2x, r2y, sel, flags = pl.pallas_call(
        _s3_kernel,
        out_shape=[jax.ShapeDtypeStruct((b, NCLS, CAP), f32)] * 6
        + [jax.ShapeDtypeStruct((b, 1, 1), f32)],
        interpret=interpret,
    )(comp, tct)

    cs4, r1x4, r1y4, r2x4, r2y4, sel4 = (
        a.reshape(b, NCLS, 1, CAP) for a in (cs3, r1x, r1y, r2x, r2y, sel))
    cls_spec = pl.BlockSpec((b, 1, 1, CAP), lambda i: (0, i, 0, 0))
    painted = pl.pallas_call(
        _s4_kernel,
        grid=(NCLS,),
        in_specs=[cls_spec] * 6,
        out_specs=pl.BlockSpec((b, 1, OUT_SZ, OUT_SZ), lambda i: (0, i, 0, 0)),
        out_shape=jax.ShapeDtypeStruct((b, NCLS, OUT_SZ, OUT_SZ), f32),
        interpret=interpret,
    )(cs4, r1x4, r1y4, r2x4, r2y4, sel4)

    out = jnp.concatenate(
        [jnp.transpose(painted, (0, 2, 3, 1)),
         jnp.zeros((b, OUT_SZ, OUT_SZ, 1), f32)], axis=3)
    return out, flags


def kernel(x):
    out_fast, flags = _run_fast(x)
    any_bad = jnp.any(flags > 0.5)

    def slow(xx):
        return jnp.where(flags.reshape(-1, 1, 1, 1) > 0.5, _run(xx), out_fast)

    return jax.lax.cond(any_bad, slow, lambda xx: out_fast, x)


# speedup vs baseline: 20.9329x; 1.1397x over previous
import functools

import jax
import jax.numpy as jnp
from jax.experimental import pallas as pl

NUM_CLASSES_K = 21
NCLS = 20
NPRIORS = 5000
OUT_SZ = 19
KEEP = 200
IOU_T = 0.45
NEG_INF = float("-inf")

CAP = 64
TGT = 48.0
JAC_IT = 8
CHUNK = 1024
NPAD = 5120
_HI = jax.lax.Precision.HIGHEST


def _decode(xt):
    loc0 = xt[0:1]
    loc1 = xt[1:2]
    loc2 = xt[2:3]
    loc3 = xt[3:4]
    p0 = xt[25:26]
    p1 = xt[26:27]
    p2 = xt[27:28]
    p3 = xt[28:29]
    v0 = xt[29:30]
    v1 = xt[30:31]
    v2 = xt[31:32]
    v3 = xt[32:33]
    pw = p2 - p0
    ph = p3 - p1
    pcx = 0.5 * (p2 + p0)
    pcy = 0.5 * (p3 + p1)
    cx = loc0 * pw * v0 + pcx
    cy = loc1 * pw * v1 + pcy
    w = jnp.exp(loc2 * v2) * pw
    h = jnp.exp(loc3 * v3) * ph
    x1 = jnp.clip(cx - 0.5 * w, 0.0, 1.0)
    y1 = jnp.clip(cy - 0.5 * h, 0.0, 1.0)
    x2 = jnp.clip(cx + 0.5 * w, 0.0, 1.0)
    y2 = jnp.clip(cy + 0.5 * h, 0.0, 1.0)
    area = (x2 - x1) * (y2 - y1)
    return x1, y1, x2, y2, area


def _paint_step(score, row, xmin, ymin, xmax, ymax, ok, acc):
    yy = jax.lax.broadcasted_iota(
        jnp.int32, (OUT_SZ, OUT_SZ, NUM_CLASSES_K), 0).astype(jnp.float32)
    xx = jax.lax.broadcasted_iota(
        jnp.int32, (OUT_SZ, OUT_SZ, NUM_CLASSES_K), 1).astype(jnp.float32)
    ch = jax.lax.broadcasted_iota(jnp.int32, (OUT_SZ, OUT_SZ, NUM_CLASSES_K), 2)
    cell = (yy >= ymin) & (yy < ymax) & (xx >= xmin) & (xx < xmax) & (ch == row)
    val = jnp.where(cell & ok, score, jnp.float32(jnp.inf))
    return jnp.minimum(acc, val)


def _nms_kernel(x_ref, out_ref):
    xt = x_ref[0]
    x1, y1, x2, y2, area = _decode(xt)
    confs = xt[5:25]
    sw0 = jnp.where(confs >= 0.6, confs, NEG_INF)
    iota_n = jax.lax.broadcasted_iota(jnp.int32, (1, NPRIORS), 1)

    def nms_body(k, carry):
        sw, ks, kx1, ky1, kx2, ky2 = carry
        m = jnp.max(sw, axis=1, keepdims=True)
        ok = m > NEG_INF
        idx = jnp.min(jnp.where(sw == m, iota_n, jnp.int32(2**30)),
                      axis=1, keepdims=True)
        pick = (iota_n == idx)
        px1 = jnp.sum(jnp.where(pick, x1, 0.0), axis=1, keepdims=True)
        py1 = jnp.sum(jnp.where(pick, y1, 0.0), axis=1, keepdims=True)
        px2 = jnp.sum(jnp.where(pick, x2, 0.0), axis=1, keepdims=True)
        py2 = jnp.sum(jnp.where(pick, y2, 0.0), axis=1, keepdims=True)
        parea = (px2 - px1) * (py2 - py1)
        xx1 = jnp.maximum(px1, x1)
        yy1 = jnp.maximum(py1, y1)
        xx2 = jnp.minimum(px2, x2)
        yy2 = jnp.minimum(py2, y2)
        inter = jnp.maximum(0.0, xx2 - xx1) * jnp.maximum(0.0, yy2 - yy1)
        union = jnp.maximum(parea + area - inter, 1e-12)
        iou = inter / union
        supp = (iou > IOU_T) | pick
        sw = jnp.where(ok & supp, NEG_INF, sw)
        mk = jnp.where(ok, m, NEG_INF)
        col = (jax.lax.broadcasted_iota(jnp.int32, (NCLS, KEEP), 1) == k)
        ks = jnp.where(col, mk, ks)
        kx1 = jnp.where(col, px1, kx1)
        ky1 = jnp.where(col, py1, ky1)
        kx2 = jnp.where(col, px2, kx2)
        ky2 = jnp.where(col, py2, ky2)
        return sw, ks, kx1, ky1, kx2, ky2

    zed = jnp.zeros((NCLS, KEEP), jnp.float32)
    ks0 = jnp.full((NCLS, KEEP), NEG_INF, jnp.float32)
    _, ks, kx1, ky1, kx2, ky2 = jax.lax.fori_loop(
        0, KEEP, nms_body, (sw0, ks0, zed, zed, zed, zed))

    scale = jnp.float32(OUT_SZ)
    rx1 = jnp.round(kx1 * scale)
    ry1 = jnp.round(ky1 * scale)
    rx2 = jnp.round(kx2 * scale)
    ry2 = jnp.round(ky2 * scale)

    flat = (jax.lax.broadcasted_iota(jnp.int32, (NCLS, KEEP), 0) * 256
            + jax.lax.broadcasted_iota(jnp.int32, (NCLS, KEEP), 1))

    def ext_body(k, carry):
        es, acc = carry
        m = jnp.max(es)
        ok = m > NEG_INF
        fk = jnp.min(jnp.where(es == m, flat, jnp.int32(2**30)))
        pick = (flat == fk)
        bx1 = jnp.sum(jnp.where(pick, rx1, 0.0))
        by1 = jnp.sum(jnp.where(pick, ry1, 0.0))
        bx2 = jnp.sum(jnp.where(pick, rx2, 0.0))
        by2 = jnp.sum(jnp.where(pick, ry2, 0.0))
        row = fk // 256
        acc = _paint_step(m, row, bx1, by1, bx2, by2, ok, acc)
        es = jnp.where(pick, NEG_INF, es)
        return es, acc

    acc0 = jnp.full((OUT_SZ, OUT_SZ, NUM_CLASSES_K), jnp.inf, jnp.float32)
    _, acc = jax.lax.fori_loop(0, KEEP, ext_body, (ks, acc0))
    out_ref[0] = jnp.where(jnp.isfinite(acc), acc, 0.0)


@functools.partial(jax.jit, static_argnames=("interpret",))
def _run(x, interpret=False):
    xt = jnp.transpose(x, (0, 2, 1))
    b = x.shape[0]
    return pl.pallas_call(
        _nms_kernel,
        grid=(b,),
        in_specs=[pl.BlockSpec((1, 33, NPRIORS), lambda i: (i, 0, 0))],
        out_specs=pl.BlockSpec((1, OUT_SZ, OUT_SZ, NUM_CLASSES_K),
                               lambda i: (i, 0, 0, 0)),
        out_shape=jax.ShapeDtypeStruct((b, OUT_SZ, OUT_SZ, NUM_CLASSES_K),
                                       jnp.float32),
        interpret=interpret,
    )(xt)


def _s1_kernel(xt_ref, cfp_ref, posx_ref, tct_ref, dx1_ref, dy1_ref,
               dx2_ref, dy2_ref):
    one = jnp.float32(1.0)
    zero = jnp.float32(0.0)
    xt = xt_ref[...]
    b = xt.shape[0]
    cf3 = xt[:, 5:25, :]
    cf = cf3.reshape(b * NCLS, NPRIORS)
    rows = cf.shape[0]

    tot_c = jnp.sum(jnp.where(cf >= 0.6, one, zero), axis=1, keepdims=True)

    def bis_body(_, lh):
        lo, hi = lh
        mid = 0.5 * (lo + hi)
        cnt = jnp.sum(jnp.where(cf >= mid, one, zero), axis=1, keepdims=True)
        ge = cnt >= TGT
        return jnp.where(ge, mid, lo), jnp.where(ge, hi, mid)

    lo0 = jnp.full((rows, 1), 0.6, jnp.float32)
    hi0 = jnp.full((rows, 1), 1.0, jnp.float32)
    tau, _ = jax.lax.fori_loop(0, 16, bis_body, (lo0, hi0))

    maskf = jnp.where(cf >= tau, one, zero)
    cnt_c = jnp.sum(maskf, axis=1, keepdims=True)

    m3 = maskf.reshape(rows, 25, 200)
    st200 = jnp.where(
        jax.lax.broadcasted_iota(jnp.int32, (200, 200), 0)
        < jax.lax.broadcasted_iota(jnp.int32, (200, 200), 1), one, zero)
    pwi = jax.lax.dot_general(m3, st200, (((2,), (0,)), ((), ())),
                              preferred_element_type=jnp.float32,
                              precision=_HI)
    tots = jnp.sum(m3, axis=2)
    st25 = jnp.where(
        jax.lax.broadcasted_iota(jnp.int32, (25, 25), 0)
        < jax.lax.broadcasted_iota(jnp.int32, (25, 25), 1), one, zero)
    offs = jax.lax.dot_general(tots, st25, (((1,), (0,)), ((), ())),
                               preferred_element_type=jnp.float32,
                               precision=_HI)
    pos = (pwi + offs[:, :, None]).reshape(rows, NPRIORS)
    posx = jnp.where(maskf > 0.5, pos, jnp.float32(-1.0))
    posx_ref[:, :, :NPRIORS] = posx.reshape(b, NCLS, NPRIORS)
    posx_ref[:, :, NPRIORS:] = jnp.full((b, NCLS, NPAD - NPRIORS), -1.0,
                                        jnp.float32)
    cfp_ref[:, :, :NPRIORS] = cf3
    cfp_ref[:, :, NPRIORS:] = jnp.zeros((b, NCLS, NPAD - NPRIORS), jnp.float32)
    tct_ref[...] = jnp.concatenate([tau, cnt_c, tot_c], axis=1)

    loc0 = xt[:, 0:1, :]
    loc1 = xt[:, 1:2, :]
    loc2 = xt[:, 2:3, :]
    loc3 = xt[:, 3:4, :]
    p0 = xt[:, 25:26, :]
    p1 = xt[:, 26:27, :]
    p2 = xt[:, 27:28, :]
    p3 = xt[:, 28:29, :]
    v0 = xt[:, 29:30, :]
    v1 = xt[:, 30:31, :]
    v2 = xt[:, 31:32, :]
    v3 = xt[:, 32:33, :]
    pw = p2 - p0
    ph = p3 - p1
    pcx = 0.5 * (p2 + p0)
    pcy = 0.5 * (p3 + p1)
    cx = loc0 * pw * v0 + pcx
    cy = loc1 * pw * v1 + pcy
    w = jnp.exp(loc2 * v2) * pw
    h = jnp.exp(loc3 * v3) * ph
    zpad = jnp.zeros((b, 1, NPAD - NPRIORS), jnp.float32)
    dx1_ref[:, :, :NPRIORS] = jnp.clip(cx - 0.5 * w, 0.0, 1.0)
    dx1_ref[:, :, NPRIORS:] = zpad
    dy1_ref[:, :, :NPRIORS] = jnp.clip(cy - 0.5 * h, 0.0, 1.0)
    dy1_ref[:, :, NPRIORS:] = zpad
    dx2_ref[:, :, :NPRIORS] = jnp.clip(cx + 0.5 * w, 0.0, 1.0)
    dx2_ref[:, :, NPRIORS:] = zpad
    dy2_ref[:, :, :NPRIORS] = jnp.clip(cy + 0.5 * h, 0.0, 1.0)
    dy2_ref[:, :, NPRIORS:] = zpad


def _s2_kernel(posx_ref, cf_ref, x1_ref, y1_ref, x2_ref, y2_ref, comp_ref):
    one = jnp.float32(1.0)
    zero = jnp.float32(0.0)

    @pl.when(pl.program_id(1) == 0)
    def _():
        comp_ref[...] = jnp.zeros_like(comp_ref)

    posm = posx_ref[0][:, None, :]
    kio = jax.lax.broadcasted_iota(jnp.int32, (1, CAP, 1), 1).astype(jnp.float32)
    ohc = jnp.where(posm == kio, one, zero)
    vals = jnp.concatenate([
        cf_ref[0][:, :, None],
        jnp.broadcast_to(x1_ref[0], (NCLS, CHUNK))[:, :, None],
        jnp.broadcast_to(y1_ref[0], (NCLS, CHUNK))[:, :, None],
        jnp.broadcast_to(x2_ref[0], (NCLS, CHUNK))[:, :, None],
        jnp.broadcast_to(y2_ref[0], (NCLS, CHUNK))[:, :, None],
    ], axis=2)
    comp_ref[...] += jax.lax.dot_general(
        ohc, vals, (((2,), (1,)), ((0,), (0,))),
        preferred_element_type=jnp.float32, precision=_HI)


def _s3_kernel(comp_ref, tct_ref, cs_ref, r1x_ref, r1y_ref, r2x_ref,
               r2y_ref, sel_ref, flag_ref):
    one = jnp.float32(1.0)
    zero = jnp.float32(0.0)
    inf = jnp.float32(jnp.inf)
    comp = comp_ref[...]
    rows = comp.shape[0]
    b = rows // NCLS
    tct = tct_ref[...]
    tau = tct[:, 0:1]
    cnt_c = tct[:, 1:2]
    tot_c = tct[:, 2:3]

    cs = comp[:, :, 0]
    cx1 = comp[:, :, 1]
    cy1 = comp[:, :, 2]
    cx2 = comp[:, :, 3]
    cy2 = comp[:, :, 4]

    slot = jax.lax.broadcasted_iota(jnp.int32, (rows, CAP), 1).astype(jnp.float32)
    validf = jnp.where(slot < cnt_c, one, zero)

    ai = (cx2 - cx1) * (cy2 - cy1)
    xx1 = jnp.maximum(cx1[:, :, None], cx1[:, None, :])
    yy1 = jnp.maximum(cy1[:, :, None], cy1[:, None, :])
    xx2 = jnp.minimum(cx2[:, :, None], cx2[:, None, :])
    yy2 = jnp.minimum(cy2[:, :, None], cy2[:, None, :])
    inter = jnp.maximum(zero, xx2 - xx1) * jnp.maximum(zero, yy2 - yy1)
    union = jnp.maximum(ai[:, :, None] + ai[:, None, :] - inter, 1e-12)
    iou = inter / union
    si = jax.lax.broadcasted_iota(jnp.int32, (1, CAP, CAP), 1)
    sj = jax.lax.broadcasted_iota(jnp.int32, (1, CAP, CAP), 2)
    higher = ((cs[:, :, None] > cs[:, None, :])
              | ((cs[:, :, None] == cs[:, None, :]) & (si < sj)))
    vpair = (validf[:, :, None] > 0.5) & (validf[:, None, :] > 0.5)
    sm = jnp.where(higher & (iou > IOU_T) & vpair, one, zero)

    def jac(k):
        sup = jnp.max(sm * k[:, :, None], axis=1)
        return jnp.where((sup < 0.5) & (validf > 0.5), one, zero)

    kcur = jax.lax.fori_loop(0, JAC_IT, lambda _, k: jac(k), validf)
    knext = jac(kcur)
    fp_ok = jnp.min(jnp.where(knext == kcur, one, zero).reshape(b, NCLS, CAP),
                    axis=(1, 2), keepdims=True)

    cs3 = cs.reshape(b, NCLS, CAP)
    kp3 = knext.reshape(b, NCLS, CAP)
    keep3 = kp3 > 0.5
    total_kept = jnp.sum(kp3, axis=(1, 2), keepdims=True)

    def b2_body(_, lh):
        lo, hi = lh
        mid = 0.5 * (lo + hi)
        cnt = jnp.sum(jnp.where(keep3 & (cs3 >= mid), one, zero),
                      axis=(1, 2), keepdims=True)
        ge = cnt >= jnp.float32(KEEP)
        return jnp.where(ge, mid, lo), jnp.where(ge, hi, mid)

    glo0 = jnp.full((b, 1, 1), 0.5, jnp.float32)
    ghi0 = jnp.full((b, 1, 1), 1.0, jnp.float32)
    glo, _ = jax.lax.fori_loop(0, 26, b2_body, (glo0, ghi0))

    tsel = keep3 & (cs3 >= glo)
    bval = jnp.min(jnp.where(tsel, cs3, inf), axis=(1, 2), keepdims=True)
    above = tsel & (cs3 > bval)
    na = jnp.sum(jnp.where(above, one, zero), axis=(1, 2), keepdims=True)
    boundary = tsel & (cs3 == bval)
    q = jnp.float32(KEEP) - na
    flat3 = (jax.lax.broadcasted_iota(jnp.int32, (1, NCLS, CAP), 1) * CAP
             + jax.lax.broadcasted_iota(jnp.int32, (1, NCLS, CAP), 2))

    def b3_body(_, lh):
        lo, hi = lh
        mid = (lo + hi + 1) // 2
        c = jnp.sum(jnp.where(boundary & (flat3 <= mid), one, zero),
                    axis=(1, 2), keepdims=True)
        le = c <= q
        return jnp.where(le, mid, lo), jnp.where(le, hi, mid - 1)

    flo0 = jnp.full((b, 1, 1), -1, jnp.int32)
    fhi0 = jnp.full((b, 1, 1), NCLS * CAP - 1, jnp.int32)
    flo, _ = jax.lax.fori_loop(0, 12, b3_body, (flo0, fhi0))

    selected = above | (boundary & (flat3 <= flo))
    self32 = jnp.where(selected, one, zero)
    sel_cnt = jnp.sum(self32, axis=(1, 2), keepdims=True)
    ext_min = jnp.min(jnp.where(selected, cs3, inf), axis=(1, 2),
                      keepdims=True)
    sel_ok = jnp.where(
        sel_cnt == jnp.minimum(total_kept, jnp.float32(KEEP)), one, zero)

    cnt2 = cnt_c.reshape(b, NCLS)
    tot2 = tot_c.reshape(b, NCLS)
    tau2 = tau.reshape(b, NCLS)
    cap_ok = jnp.min(jnp.where(cnt2 <= jnp.float32(CAP), one, zero),
                     axis=1, keepdims=True)[:, :, None]
    fullf = jnp.where(cnt2 == tot2, one, zero)
    all_full = jnp.min(fullf, axis=1, keepdims=True)[:, :, None]
    emin2 = ext_min.reshape(b, 1)
    complete = jnp.min(
        jnp.where((fullf > 0.5) | (tau2 <= emin2), one, zero),
        axis=1, keepdims=True)[:, :, None]
    exact = (cap_ok * fp_ok * sel_ok
             * jnp.where(sel_cnt == jnp.float32(KEEP), complete, all_full))
    flag_ref[...] = one - jnp.minimum(exact, one)

    scale = jnp.float32(OUT_SZ)
    cs_ref[...] = cs3
    r1x_ref[...] = jnp.round(cx1 * scale).reshape(b, NCLS, CAP)
    r1y_ref[...] = jnp.round(cy1 * scale).reshape(b, NCLS, CAP)
    r2x_ref[...] = jnp.round(cx2 * scale).reshape(b, NCLS, CAP)
    r2y_ref[...] = jnp.round(cy2 * scale).reshape(b, NCLS, CAP)
    sel_ref[...] = self32


def _s4_kernel(cs_ref, r1x_ref, r1y_ref, r2x_ref, r2y_ref, sel_ref, out_ref):
    zero = jnp.float32(0.0)
    inf = jnp.float32(jnp.inf)
    yy = jax.lax.broadcasted_iota(
        jnp.int32, (1, OUT_SZ, OUT_SZ, CAP), 1).astype(jnp.float32)
    xx = jax.lax.broadcasted_iota(
        jnp.int32, (1, OUT_SZ, OUT_SZ, CAP), 2).astype(jnp.float32)
    ry1c = r1y_ref[...]
    ry2c = r2y_ref[...]
    rx1c = r1x_ref[...]
    rx2c = r2x_ref[...]
    selc = sel_ref[...]
    csc = cs_ref[...]
    cover = ((yy >= ry1c) & (yy < ry2c) & (xx >= rx1c) & (xx < rx2c)
             & (selc > 0.5))
    vals = jnp.where(cover, csc, inf)
    acc = jnp.min(vals, axis=3)
    out_ref[...] = jnp.where(jnp.isfinite(acc), acc, zero)[:, None, :, :]


@functools.partial(jax.jit, static_argnames=("interpret",))
def _run_fast(x, interpret=False):
    b = x.shape[0]
    rows = b * NCLS
    xt = jnp.transpose(x, (0, 2, 1))

    f32 = jnp.float32
    cfp, posx, tct, dx1, dy1, dx2, dy2 = pl.pallas_call(
        _s1_kernel,
        out_shape=[
            jax.ShapeDtypeStruct((b, NCLS, NPAD), f32),
            jax.ShapeDtypeStruct((b, NCLS, NPAD), f32),
            jax.ShapeDtypeStruct((rows, 3), f32),
            jax.ShapeDtypeStruct((b, 1, NPAD), f32),
            jax.ShapeDtypeStruct((b, 1, NPAD), f32),
            jax.ShapeDtypeStruct((b, 1, NPAD), f32),
            jax.ShapeDtypeStruct((b, 1, NPAD), f32),
        ],
        interpret=interpret,
    )(xt)

    nchunks = NPAD // CHUNK
    row_spec = pl.BlockSpec((1, NCLS, CHUNK), lambda j, i: (j, 0, i))
    box_spec = pl.BlockSpec((1, 1, CHUNK), lambda j, i: (j, 0, i))
    comp = pl.pallas_call(
        _s2_kernel,
        grid=(b, nchunks),
        in_specs=[row_spec, row_spec, box_spec, box_spec, box_spec, box_spec],
        out_specs=pl.BlockSpec((NCLS, CAP, 5), lambda j, i: (j, 0, 0)),
        out_shape=jax.ShapeDtypeStruct((rows, CAP, 5), f32),
        interpret=interpret,
    )(posx, cfp, dx1, dy1, dx2, dy2)

    cs3, r1x, r1y, r2x, r2y, sel, flags = pl.pallas_call(
        _s3_kernel,
        out_shape=[jax.ShapeDtypeStruct((b, NCLS, CAP), f32)] * 6
        + [jax.ShapeDtypeStruct((b, 1, 1), f32)],
        interpret=interpret,
    )(comp, tct)

    cs4, r1x4, r1y4, r2x4, r2y4, sel4 = (
        a.reshape(b, NCLS, 1, CAP) for a in (cs3, r1x, r1y, r2x, r2y, sel))
    cls_spec = pl.BlockSpec((b, 1, 1, CAP), lambda i: (0, i, 0, 0))
    painted = pl.pallas_call(
        _s4_kernel,
        grid=(NCLS,),
        in_specs=[cls_spec] * 6,
        out_specs=pl.BlockSpec((b, 1, OUT_SZ, OUT_SZ), lambda i: (0, i, 0, 0)),
        out_shape=jax.ShapeDtypeStruct((b, NCLS, OUT_SZ, OUT_SZ), f32),
        interpret=interpret,
    )(cs4, r1x4, r1y4, r2x4, r2y4, sel4)

    out = jnp.concatenate(
        [jnp.transpose(painted, (0, 2, 3, 1)),
         jnp.zeros((b, OUT_SZ, OUT_SZ, 1), f32)], axis=3)
    return out, flags


def kernel(x):
    out_fast, flags = _run_fast(x)
    any_bad = jnp.any(flags > 0.5)

    def slow(xx):
        return jnp.where(flags.reshape(-1, 1, 1, 1) > 0.5, _run(xx), out_fast)

    return jax.lax.cond(any_bad, slow, lambda xx: out_fast, x)
